# Initial kernel scaffold; baseline (speedup 1.0000x reference)
import functools

import jax
import jax.numpy as jnp
from jax import lax
from jax.experimental import pallas as pl
from jax.experimental.pallas import tpu as pltpu
from jax.experimental.pallas import tpu_sc as plsc


CHUNK = 128


def _make_scalar_scatter(NW, NC, NS, CPW, ACC_R):
    RPT = ACC_R // NS
    mesh = plsc.VectorSubcoreMesh(core_axis_name="c", subcore_axis_name="s")

    @functools.partial(
        pl.kernel,
        mesh=mesh,
        out_type=jax.ShapeDtypeStruct((NC, ACC_R), jnp.float32),
        scratch_types=[
            pltpu.VMEM((CPW, CHUNK), jnp.int32),
            pltpu.VMEM((CPW, CHUNK), jnp.int32),
            pltpu.VMEM((CHUNK,), jnp.float32),
            pltpu.VMEM_SHARED((ACC_R,), jnp.float32),
            pltpu.SemaphoreType.DMA,
        ],
    )
    def scatter_kernel(src_hbm, dst_hbm, vals_hbm, zeros_hbm, out_hbm,
                       idx_src_v, idx_dst_v, vals_v, acc_sh, sem):
        c = lax.axis_index("c")
        s = lax.axis_index("s")
        wid = s * NC + c
        pltpu.sync_copy(zeros_hbm.at[pl.ds(s * RPT, RPT)],
                        acc_sh.at[pl.ds(s * RPT, RPT)])
        pltpu.sync_copy(src_hbm.at[wid], idx_src_v)
        pltpu.sync_copy(dst_hbm.at[wid], idx_dst_v)
        plsc.subcore_barrier()

        @pl.loop(0, CPW)
        def _(j):
            pltpu.async_copy(vals_hbm.at[idx_src_v.at[j]], vals_v, sem).wait()
            pltpu.sync_copy(vals_v, acc_sh.at[idx_dst_v.at[j]], add=True)

        plsc.subcore_barrier()
        pltpu.sync_copy(acc_sh.at[pl.ds(s * RPT, RPT)],
                        out_hbm.at[c, pl.ds(s * RPT, RPT)])

    return scatter_kernel


def _make_row_scatter(NW, NC, NS, CPW, ACC_R, H):
    RPT = ACC_R // NS
    mesh = plsc.VectorSubcoreMesh(core_axis_name="c", subcore_axis_name="s")

    @functools.partial(
        pl.kernel,
        mesh=mesh,
        out_type=jax.ShapeDtypeStruct((NC, ACC_R, H), jnp.float32),
        scratch_types=[
            pltpu.VMEM((CPW, CHUNK), jnp.int32),
            pltpu.VMEM((CPW, CHUNK), jnp.int32),
            pltpu.VMEM((CHUNK, H), jnp.float32),
            pltpu.VMEM_SHARED((ACC_R, H), jnp.float32),
            pltpu.SemaphoreType.DMA,
        ],
    )
    def row_scatter_kernel(src_hbm, dst_hbm, g1_hbm, zeros_hbm, out_hbm,
                           idx_src_v, idx_dst_v, rows_v, acc_sh, sem):
        c = lax.axis_index("c")
        s = lax.axis_index("s")
        wid = s * NC + c
        pltpu.sync_copy(zeros_hbm.at[pl.ds(s * RPT, RPT)],
                        acc_sh.at[pl.ds(s * RPT, RPT)])
        pltpu.sync_copy(src_hbm.at[wid], idx_src_v)
        pltpu.sync_copy(dst_hbm.at[wid], idx_dst_v)
        plsc.subcore_barrier()

        @pl.loop(0, CPW)
        def _(j):
            pltpu.async_copy(g1_hbm.at[idx_src_v.at[j]], rows_v, sem).wait()
            pltpu.sync_copy(rows_v, acc_sh.at[idx_dst_v.at[j]], add=True)

        plsc.subcore_barrier()
        pltpu.sync_copy(acc_sh.at[pl.ds(s * RPT, RPT)],
                        out_hbm.at[c, pl.ds(s * RPT, RPT)])

    return row_scatter_kernel


def _tc1_body(x_ref, w1_ref, c0_ref, c1_ref, g1_ref, dis_ref):
    deg = 1.0 + c0_ref[...] + c1_ref[...]
    dis = lax.rsqrt(deg)
    h1 = jnp.dot(x_ref[...], w1_ref[...], preferred_element_type=jnp.float32)
    g1_ref[...] = h1 * dis
    dis_ref[...] = dis


def _tc2_body(dis_ref, g1_ref, p0_ref, p1_ref, b1_ref, w2_ref, g2_ref):
    dis = dis_ref[...]
    out1 = dis * (p0_ref[...] + p1_ref[...] + g1_ref[...]) + b1_ref[...]
    z = jnp.maximum(out1, 0.0)
    h2 = jnp.dot(z, w2_ref[...], preferred_element_type=jnp.float32)
    g2_ref[...] = dis * h2


def _tc3_body(dis_ref, g2_ref, q0_ref, q1_ref, b2_ref, out_ref):
    out_ref[...] = (dis_ref[...] * (q0_ref[...] + q1_ref[...] + g2_ref[...])
                    + b2_ref[...])


def kernel(x, edge_index, W1, b1, W2, b2):
    N, D = x.shape
    H = W1.shape[1]
    O = W2.shape[1]
    E = edge_index.shape[1]

    info = plsc.get_sparse_core_info()
    NC, NS = info.num_cores, info.num_subcores
    NW = NC * NS
    CPW = -(-E // (NW * CHUNK))
    EPAD = NW * CPW * CHUNK
    ACC_R = ((N + NS * 8 - 1) // (NS * 8)) * (NS * 8)
    if ACC_R == N:
        ACC_R += NS * 8
    pad = EPAD - E

    src = edge_index[0]
    dst = edge_index[1]
    pad_i = jnp.arange(pad, dtype=jnp.int32)
    srcp = jnp.concatenate([src, pad_i % N]).reshape(NW, CPW, CHUNK)
    dstp = jnp.concatenate([dst, N + pad_i % (ACC_R - N)]).reshape(NW, CPW, CHUNK)

    zeros2 = jnp.zeros((ACC_R, H), jnp.float32)
    zeros1 = jnp.zeros((ACC_R,), jnp.float32)
    ones_n = jnp.ones((N,), jnp.float32)

    scalar_scatter = _make_scalar_scatter(NW, NC, NS, CPW, ACC_R)
    row_scatter = _make_row_scatter(NW, NC, NS, CPW, ACC_R, H)

    cnt = scalar_scatter(srcp, dstp, ones_n, zeros1)
    c0 = cnt[0, :N, None]
    c1 = cnt[1, :N, None]

    BN = 2000
    grid = (N // BN,)
    g1, dis = pl.pallas_call(
        _tc1_body,
        grid=grid,
        in_specs=[
            pl.BlockSpec((BN, D), lambda i: (i, 0)),
            pl.BlockSpec((D, H), lambda i: (0, 0)),
            pl.BlockSpec((BN, 1), lambda i: (i, 0)),
            pl.BlockSpec((BN, 1), lambda i: (i, 0)),
        ],
        out_specs=[
            pl.BlockSpec((BN, H), lambda i: (i, 0)),
            pl.BlockSpec((BN, 1), lambda i: (i, 0)),
        ],
        out_shape=[
            jax.ShapeDtypeStruct((N, H), jnp.float32),
            jax.ShapeDtypeStruct((N, 1), jnp.float32),
        ],
    )(x, W1, c0, c1)

    p = row_scatter(srcp, dstp, g1, zeros2)
    p0 = p[0, :N, :]
    p1 = p[1, :N, :]

    g2 = pl.pallas_call(
        _tc2_body,
        grid=grid,
        in_specs=[
            pl.BlockSpec((BN, 1), lambda i: (i, 0)),
            pl.BlockSpec((BN, H), lambda i: (i, 0)),
            pl.BlockSpec((BN, H), lambda i: (i, 0)),
            pl.BlockSpec((BN, H), lambda i: (i, 0)),
            pl.BlockSpec((1, H), lambda i: (0, 0)),
            pl.BlockSpec((H, O), lambda i: (0, 0)),
        ],
        out_specs=pl.BlockSpec((BN, O), lambda i: (i, 0)),
        out_shape=jax.ShapeDtypeStruct((N, O), jnp.float32),
    )(dis, g1, p0, p1, b1[None, :], W2)

    q = scalar_scatter(srcp, dstp, g2[:, 0], zeros1)
    q0 = q[0, :N, None]
    q1 = q[1, :N, None]

    out = pl.pallas_call(
        _tc3_body,
        grid=grid,
        in_specs=[
            pl.BlockSpec((BN, 1), lambda i: (i, 0)),
            pl.BlockSpec((BN, O), lambda i: (i, 0)),
            pl.BlockSpec((BN, 1), lambda i: (i, 0)),
            pl.BlockSpec((BN, 1), lambda i: (i, 0)),
            pl.BlockSpec((1, 1), lambda i: (0, 0)),
        ],
        out_specs=pl.BlockSpec((BN, O), lambda i: (i, 0)),
        out_shape=jax.ShapeDtypeStruct((N, O), jnp.float32),
    )(dis, g2, q0, q1, b2[None, :])

    return out


# speedup vs baseline: 28.2403x; 28.2403x over previous
import functools

import jax
import jax.numpy as jnp
from jax import lax
from jax.experimental import pallas as pl
from jax.experimental.pallas import tpu as pltpu
from jax.experimental.pallas import tpu_sc as plsc


CHUNK = 128


def _make_scalar_scatter(NW, NC, NS, CPW, ACC_R):
    RPT = ACC_R // NS
    mesh = plsc.VectorSubcoreMesh(core_axis_name="c", subcore_axis_name="s")

    @functools.partial(
        pl.kernel,
        mesh=mesh,
        compiler_params=pltpu.CompilerParams(use_tc_tiling_on_sc=False),
        out_type=jax.ShapeDtypeStruct((NC * ACC_R,), jnp.float32),
        scratch_types=[
            pltpu.VMEM((CPW, CHUNK), jnp.int32),
            pltpu.VMEM((CPW, CHUNK), jnp.int32),
            pltpu.VMEM((CHUNK,), jnp.float32),
            pltpu.VMEM_SHARED((ACC_R,), jnp.float32),
            pltpu.SemaphoreType.DMA,
        ],
    )
    def scatter_kernel(src_hbm, dst_hbm, vals_hbm, zeros_hbm, out_hbm,
                       idx_src_v, idx_dst_v, vals_v, acc_sh, sem):
        c = lax.axis_index("c")
        s = lax.axis_index("s")
        wid = s * NC + c
        pltpu.sync_copy(zeros_hbm.at[pl.ds(s * RPT, RPT)],
                        acc_sh.at[pl.ds(s * RPT, RPT)])
        pltpu.sync_copy(src_hbm.at[wid], idx_src_v)
        pltpu.sync_copy(dst_hbm.at[wid], idx_dst_v)
        plsc.subcore_barrier()

        @pl.loop(0, CPW)
        def _(j):
            pltpu.async_copy(vals_hbm.at[idx_src_v.at[j]], vals_v, sem).wait()
            pltpu.sync_copy(vals_v, acc_sh.at[idx_dst_v.at[j]], add=True)

        plsc.subcore_barrier()
        pltpu.sync_copy(acc_sh.at[pl.ds(s * RPT, RPT)],
                        out_hbm.at[pl.ds(c * ACC_R + s * RPT, RPT)])

    return scatter_kernel


def _make_row_scatter(NW, NC, NS, CPW, ACC_R, H):
    RPT = ACC_R // NS
    mesh = plsc.VectorSubcoreMesh(core_axis_name="c", subcore_axis_name="s")

    @functools.partial(
        pl.kernel,
        mesh=mesh,
        compiler_params=pltpu.CompilerParams(use_tc_tiling_on_sc=False),
        out_type=jax.ShapeDtypeStruct((NC, ACC_R, H), jnp.float32),
        scratch_types=[
            pltpu.VMEM((CPW, CHUNK), jnp.int32),
            pltpu.VMEM((CPW, CHUNK), jnp.int32),
            pltpu.VMEM((CHUNK, H), jnp.float32),
            pltpu.VMEM_SHARED((ACC_R, H), jnp.float32),
            pltpu.SemaphoreType.DMA,
        ],
    )
    def row_scatter_kernel(src_hbm, dst_hbm, g1_hbm, zeros_hbm, out_hbm,
                           idx_src_v, idx_dst_v, rows_v, acc_sh, sem):
        c = lax.axis_index("c")
        s = lax.axis_index("s")
        wid = s * NC + c
        pltpu.sync_copy(zeros_hbm.at[pl.ds(s * RPT, RPT)],
                        acc_sh.at[pl.ds(s * RPT, RPT)])
        pltpu.sync_copy(src_hbm.at[wid], idx_src_v)
        pltpu.sync_copy(dst_hbm.at[wid], idx_dst_v)
        plsc.subcore_barrier()

        @pl.loop(0, CPW)
        def _(j):
            pltpu.async_copy(g1_hbm.at[idx_src_v.at[j]], rows_v, sem).wait()
            pltpu.sync_copy(rows_v, acc_sh.at[idx_dst_v.at[j]], add=True)

        plsc.subcore_barrier()
        pltpu.sync_copy(acc_sh.at[pl.ds(s * RPT, RPT)],
                        out_hbm.at[c].at[pl.ds(s * RPT, RPT)])

    return row_scatter_kernel


def _tc1_body(x_ref, w1_ref, c0_ref, c1_ref, g1_ref, dis_ref):
    deg = 1.0 + c0_ref[...] + c1_ref[...]
    dis = lax.rsqrt(deg)
    h1 = jnp.dot(x_ref[...], w1_ref[...], preferred_element_type=jnp.float32)
    g1_ref[...] = h1 * dis
    dis_ref[...] = dis


def _tc2_body(dis_ref, g1_ref, p0_ref, p1_ref, b1_ref, w2_ref, g2_ref):
    dis = dis_ref[...]
    out1 = dis * (p0_ref[...] + p1_ref[...] + g1_ref[...]) + b1_ref[...]
    z = jnp.maximum(out1, 0.0)
    h2 = jnp.dot(z, w2_ref[...], preferred_element_type=jnp.float32)
    g2_ref[...] = dis * h2


def _tc3_body(dis_ref, g2_ref, q0_ref, q1_ref, b2_ref, out_ref):
    out_ref[...] = (dis_ref[...] * (q0_ref[...] + q1_ref[...] + g2_ref[...])
                    + b2_ref[...])


def kernel(x, edge_index, W1, b1, W2, b2):
    N, D = x.shape
    H = W1.shape[1]
    O = W2.shape[1]
    E = edge_index.shape[1]

    info = plsc.get_sparse_core_info()
    NC, NS = info.num_cores, info.num_subcores
    NW = NC * NS
    CPW = -(-E // (NW * CHUNK))
    CPW += CPW % 2
    EPAD = NW * CPW * CHUNK
    ACC_R = ((N + NS * 128 - 1) // (NS * 128)) * (NS * 128)
    if ACC_R == N:
        ACC_R += NS * 128
    pad = EPAD - E

    src = edge_index[0]
    dst = edge_index[1]
    pad_i = jnp.arange(pad, dtype=jnp.int32)
    srcp = jnp.concatenate([src, pad_i % N]).reshape(NW, CPW, CHUNK)
    dstp = jnp.concatenate([dst, N + pad_i % (ACC_R - N)]).reshape(NW, CPW, CHUNK)

    zeros2 = jnp.zeros((ACC_R, H), jnp.float32)
    zeros1 = jnp.zeros((ACC_R,), jnp.float32)
    ones_n = jnp.ones((N,), jnp.float32)

    scalar_scatter = _make_scalar_scatter(NW, NC, NS, CPW, ACC_R)
    row_scatter = _make_row_scatter(NW, NC, NS, CPW, ACC_R, H)

    cnt = scalar_scatter(srcp, dstp, ones_n, zeros1).reshape(NC, ACC_R)
    c0 = cnt[0, :N, None]
    c1 = cnt[1, :N, None]

    BN = 2000
    grid = (N // BN,)
    g1, dis = pl.pallas_call(
        _tc1_body,
        grid=grid,
        in_specs=[
            pl.BlockSpec((BN, D), lambda i: (i, 0)),
            pl.BlockSpec((D, H), lambda i: (0, 0)),
            pl.BlockSpec((BN, 1), lambda i: (i, 0)),
            pl.BlockSpec((BN, 1), lambda i: (i, 0)),
        ],
        out_specs=[
            pl.BlockSpec((BN, H), lambda i: (i, 0)),
            pl.BlockSpec((BN, 1), lambda i: (i, 0)),
        ],
        out_shape=[
            jax.ShapeDtypeStruct((N, H), jnp.float32),
            jax.ShapeDtypeStruct((N, 1), jnp.float32),
        ],
    )(x, W1, c0, c1)

    p = row_scatter(srcp, dstp, g1, zeros2)
    p0 = p[0, :N, :]
    p1 = p[1, :N, :]

    g2 = pl.pallas_call(
        _tc2_body,
        grid=grid,
        in_specs=[
            pl.BlockSpec((BN, 1), lambda i: (i, 0)),
            pl.BlockSpec((BN, H), lambda i: (i, 0)),
            pl.BlockSpec((BN, H), lambda i: (i, 0)),
            pl.BlockSpec((BN, H), lambda i: (i, 0)),
            pl.BlockSpec((1, H), lambda i: (0, 0)),
            pl.BlockSpec((H, O), lambda i: (0, 0)),
        ],
        out_specs=pl.BlockSpec((BN, O), lambda i: (i, 0)),
        out_shape=jax.ShapeDtypeStruct((N, O), jnp.float32),
    )(dis, g1, p0, p1, b1[None, :], W2)

    q = scalar_scatter(srcp, dstp, g2[:, 0], zeros1).reshape(NC, ACC_R)
    q0 = q[0, :N, None]
    q1 = q[1, :N, None]

    out = pl.pallas_call(
        _tc3_body,
        grid=grid,
        in_specs=[
            pl.BlockSpec((BN, 1), lambda i: (i, 0)),
            pl.BlockSpec((BN, O), lambda i: (i, 0)),
            pl.BlockSpec((BN, 1), lambda i: (i, 0)),
            pl.BlockSpec((BN, 1), lambda i: (i, 0)),
            pl.BlockSpec((1, 1), lambda i: (0, 0)),
        ],
        out_specs=pl.BlockSpec((BN, O), lambda i: (i, 0)),
        out_shape=jax.ShapeDtypeStruct((N, O), jnp.float32),
    )(dis, g2, q0, q1, b2[None, :])

    return out


# speedup vs baseline: 38.7479x; 1.3721x over previous
import functools

import jax
import jax.numpy as jnp
from jax import lax
from jax.experimental import pallas as pl
from jax.experimental.pallas import tpu as pltpu
from jax.experimental.pallas import tpu_sc as plsc


CHUNK = 128
WAVE = 8


def _make_hist(NW, NC, NS, CPW, ACC_R):
    RPT = ACC_R // NS
    mesh = plsc.VectorSubcoreMesh(core_axis_name="c", subcore_axis_name="s")

    @functools.partial(
        pl.kernel,
        mesh=mesh,
        compiler_params=pltpu.CompilerParams(use_tc_tiling_on_sc=False),
        out_type=jax.ShapeDtypeStruct((NC * ACC_R,), jnp.float32),
        scratch_types=[
            pltpu.VMEM((CPW, CHUNK), jnp.int32),
            pltpu.VMEM((CHUNK,), jnp.float32),
            pltpu.VMEM_SHARED((ACC_R,), jnp.float32),
            pltpu.SemaphoreType.DMA,
        ],
    )
    def hist_kernel(dst_hbm, ones_hbm, zeros_hbm, out_hbm,
                    idx_dst_v, ones_v, acc_sh, sem):
        c = lax.axis_index("c")
        s = lax.axis_index("s")
        wid = s * NC + c
        pltpu.sync_copy(zeros_hbm.at[pl.ds(s * RPT, RPT)],
                        acc_sh.at[pl.ds(s * RPT, RPT)])
        pltpu.sync_copy(dst_hbm.at[wid], idx_dst_v)
        pltpu.sync_copy(ones_hbm, ones_v)
        plsc.subcore_barrier()

        @pl.loop(0, CPW, step=WAVE)
        def _(j0):
            for b in range(WAVE):
                pltpu.async_copy(ones_v, acc_sh.at[idx_dst_v.at[j0 + b]],
                                 sem, add=True)
            for b in range(WAVE):
                pltpu.make_async_copy(ones_v, acc_sh.at[idx_dst_v.at[j0 + b]],
                                      sem).wait()

        plsc.subcore_barrier()
        pltpu.sync_copy(acc_sh.at[pl.ds(s * RPT, RPT)],
                        out_hbm.at[pl.ds(c * ACC_R + s * RPT, RPT)])

    return hist_kernel


def _make_scalar_scatter(NW, NC, NS, CPW, ACC_R):
    RPT = ACC_R // NS
    mesh = plsc.VectorSubcoreMesh(core_axis_name="c", subcore_axis_name="s")

    @functools.partial(
        pl.kernel,
        mesh=mesh,
        compiler_params=pltpu.CompilerParams(use_tc_tiling_on_sc=False),
        out_type=jax.ShapeDtypeStruct((NC * ACC_R,), jnp.float32),
        scratch_types=[
            pltpu.VMEM((CPW, CHUNK), jnp.int32),
            pltpu.VMEM((CPW, CHUNK), jnp.int32),
            pltpu.VMEM((2, CHUNK), jnp.float32),
            pltpu.VMEM_SHARED((ACC_R,), jnp.float32),
            pltpu.SemaphoreType.DMA,
        ],
    )
    def scatter_kernel(src_hbm, dst_hbm, vals_hbm, zeros_hbm, out_hbm,
                       idx_src_v, idx_dst_v, vals_v, acc_sh, sem):
        c = lax.axis_index("c")
        s = lax.axis_index("s")
        wid = s * NC + c
        pltpu.sync_copy(zeros_hbm.at[pl.ds(s * RPT, RPT)],
                        acc_sh.at[pl.ds(s * RPT, RPT)])
        pltpu.sync_copy(src_hbm.at[wid], idx_src_v)
        pltpu.sync_copy(dst_hbm.at[wid], idx_dst_v)
        plsc.subcore_barrier()

        pltpu.async_copy(vals_hbm.at[idx_src_v.at[0]], vals_v.at[0], sem)

        @pl.loop(0, CPW, step=2)
        def _(j):
            for b in range(2):
                jj = j + b
                pltpu.make_async_copy(vals_hbm.at[idx_src_v.at[jj]],
                                      vals_v.at[b], sem).wait()

                @pl.when(jj + 1 < CPW)
                def _fire():
                    pltpu.async_copy(vals_hbm.at[idx_src_v.at[jj + 1]],
                                     vals_v.at[1 - b], sem)

                pltpu.sync_copy(vals_v.at[b], acc_sh.at[idx_dst_v.at[jj]],
                                add=True)

        plsc.subcore_barrier()
        pltpu.sync_copy(acc_sh.at[pl.ds(s * RPT, RPT)],
                        out_hbm.at[pl.ds(c * ACC_R + s * RPT, RPT)])

    return scatter_kernel


def _make_row_scatter(NW, NC, NS, CPW, ACC_R, H):
    RPT = ACC_R // NS
    mesh = plsc.VectorSubcoreMesh(core_axis_name="c", subcore_axis_name="s")

    @functools.partial(
        pl.kernel,
        mesh=mesh,
        compiler_params=pltpu.CompilerParams(use_tc_tiling_on_sc=False),
        out_type=jax.ShapeDtypeStruct((NC, ACC_R, H), jnp.float32),
        scratch_types=[
            pltpu.VMEM((CPW, CHUNK), jnp.int32),
            pltpu.VMEM((CPW, CHUNK), jnp.int32),
            pltpu.VMEM((2, CHUNK, H), jnp.float32),
            pltpu.VMEM_SHARED((ACC_R, H), jnp.float32),
            pltpu.SemaphoreType.DMA,
        ],
    )
    def row_scatter_kernel(src_hbm, dst_hbm, g1_hbm, zeros_hbm, out_hbm,
                           idx_src_v, idx_dst_v, rows_v, acc_sh, sem):
        c = lax.axis_index("c")
        s = lax.axis_index("s")
        wid = s * NC + c
        pltpu.sync_copy(zeros_hbm.at[pl.ds(s * RPT, RPT)],
                        acc_sh.at[pl.ds(s * RPT, RPT)])
        pltpu.sync_copy(src_hbm.at[wid], idx_src_v)
        pltpu.sync_copy(dst_hbm.at[wid], idx_dst_v)
        plsc.subcore_barrier()

        pltpu.async_copy(g1_hbm.at[idx_src_v.at[0]], rows_v.at[0], sem)

        @pl.loop(0, CPW, step=2)
        def _(j):
            for b in range(2):
                jj = j + b
                pltpu.make_async_copy(g1_hbm.at[idx_src_v.at[jj]],
                                      rows_v.at[b], sem).wait()

                @pl.when(jj + 1 < CPW)
                def _fire():
                    pltpu.async_copy(g1_hbm.at[idx_src_v.at[jj + 1]],
                                     rows_v.at[1 - b], sem)

                pltpu.sync_copy(rows_v.at[b], acc_sh.at[idx_dst_v.at[jj]],
                                add=True)

        plsc.subcore_barrier()
        pltpu.sync_copy(acc_sh.at[pl.ds(s * RPT, RPT)],
                        out_hbm.at[c].at[pl.ds(s * RPT, RPT)])

    return row_scatter_kernel


def _tc1_body(x_ref, w1_ref, c0_ref, c1_ref, g1_ref, dis_ref):
    deg = 1.0 + c0_ref[...] + c1_ref[...]
    dis = lax.rsqrt(deg)
    h1 = jnp.dot(x_ref[...], w1_ref[...], preferred_element_type=jnp.float32)
    g1_ref[...] = h1 * dis
    dis_ref[...] = dis


def _tc2_body(dis_ref, g1_ref, p0_ref, p1_ref, b1_ref, w2_ref, g2_ref):
    dis = dis_ref[...]
    out1 = dis * (p0_ref[...] + p1_ref[...] + g1_ref[...]) + b1_ref[...]
    z = jnp.maximum(out1, 0.0)
    h2 = jnp.dot(z, w2_ref[...], preferred_element_type=jnp.float32)
    g2_ref[...] = dis * h2


def _tc3_body(dis_ref, g2_ref, q0_ref, q1_ref, b2_ref, out_ref):
    out_ref[...] = (dis_ref[...] * (q0_ref[...] + q1_ref[...] + g2_ref[...])
                    + b2_ref[...])


def kernel(x, edge_index, W1, b1, W2, b2):
    N, D = x.shape
    H = W1.shape[1]
    O = W2.shape[1]
    E = edge_index.shape[1]

    info = plsc.get_sparse_core_info()
    NC, NS = info.num_cores, info.num_subcores
    NW = NC * NS
    CPW = -(-E // (NW * CHUNK))
    CPW += CPW % 2
    EPAD = NW * CPW * CHUNK
    ACC_R = ((N + NS * 128 - 1) // (NS * 128)) * (NS * 128)
    if ACC_R == N:
        ACC_R += NS * 128
    pad = EPAD - E

    src = edge_index[0]
    dst = edge_index[1]
    pad_i = jnp.arange(pad, dtype=jnp.int32)
    srcp = jnp.concatenate([src, pad_i % N]).reshape(NW, CPW, CHUNK)
    dstp = jnp.concatenate([dst, N + pad_i % (ACC_R - N)]).reshape(NW, CPW, CHUNK)

    zeros2 = jnp.zeros((ACC_R, H), jnp.float32)
    zeros1 = jnp.zeros((ACC_R,), jnp.float32)
    ones_c = jnp.ones((CHUNK,), jnp.float32)

    hist = _make_hist(NW, NC, NS, CPW, ACC_R)
    scalar_scatter = _make_scalar_scatter(NW, NC, NS, CPW, ACC_R)
    row_scatter = _make_row_scatter(NW, NC, NS, CPW, ACC_R, H)

    cnt = hist(dstp, ones_c, zeros1).reshape(NC, ACC_R)
    c0 = cnt[0, :N, None]
    c1 = cnt[1, :N, None]

    BN = 2000
    grid = (N // BN,)
    g1, dis = pl.pallas_call(
        _tc1_body,
        grid=grid,
        in_specs=[
            pl.BlockSpec((BN, D), lambda i: (i, 0)),
            pl.BlockSpec((D, H), lambda i: (0, 0)),
            pl.BlockSpec((BN, 1), lambda i: (i, 0)),
            pl.BlockSpec((BN, 1), lambda i: (i, 0)),
        ],
        out_specs=[
            pl.BlockSpec((BN, H), lambda i: (i, 0)),
            pl.BlockSpec((BN, 1), lambda i: (i, 0)),
        ],
        out_shape=[
            jax.ShapeDtypeStruct((N, H), jnp.float32),
            jax.ShapeDtypeStruct((N, 1), jnp.float32),
        ],
    )(x, W1, c0, c1)

    p = row_scatter(srcp, dstp, g1, zeros2)
    p0 = p[0, :N, :]
    p1 = p[1, :N, :]

    g2 = pl.pallas_call(
        _tc2_body,
        grid=grid,
        in_specs=[
            pl.BlockSpec((BN, 1), lambda i: (i, 0)),
            pl.BlockSpec((BN, H), lambda i: (i, 0)),
            pl.BlockSpec((BN, H), lambda i: (i, 0)),
            pl.BlockSpec((BN, H), lambda i: (i, 0)),
            pl.BlockSpec((1, H), lambda i: (0, 0)),
            pl.BlockSpec((H, O), lambda i: (0, 0)),
        ],
        out_specs=pl.BlockSpec((BN, O), lambda i: (i, 0)),
        out_shape=jax.ShapeDtypeStruct((N, O), jnp.float32),
    )(dis, g1, p0, p1, b1[None, :], W2)

    q = scalar_scatter(srcp, dstp, g2[:, 0], zeros1).reshape(NC, ACC_R)
    q0 = q[0, :N, None]
    q1 = q[1, :N, None]

    out = pl.pallas_call(
        _tc3_body,
        grid=grid,
        in_specs=[
            pl.BlockSpec((BN, 1), lambda i: (i, 0)),
            pl.BlockSpec((BN, O), lambda i: (i, 0)),
            pl.BlockSpec((BN, 1), lambda i: (i, 0)),
            pl.BlockSpec((BN, 1), lambda i: (i, 0)),
            pl.BlockSpec((1, 1), lambda i: (0, 0)),
        ],
        out_specs=pl.BlockSpec((BN, O), lambda i: (i, 0)),
        out_shape=jax.ShapeDtypeStruct((N, O), jnp.float32),
    )(dis, g2, q0, q1, b2[None, :])

    return out


# speedup vs baseline: 63.2345x; 1.6319x over previous
import functools

import jax
import jax.numpy as jnp
from jax import lax
from jax.experimental import pallas as pl
from jax.experimental.pallas import tpu as pltpu
from jax.experimental.pallas import tpu_sc as plsc


CHUNK = 128
WAVE = 8
RWAVE = 4


def _make_hist(NW, NC, NS, CPW, ACC_R):
    RPT = ACC_R // NS
    mesh = plsc.VectorSubcoreMesh(core_axis_name="c", subcore_axis_name="s")

    @functools.partial(
        pl.kernel,
        mesh=mesh,
        compiler_params=pltpu.CompilerParams(use_tc_tiling_on_sc=False,
                                             needs_layout_passes=False),
        out_type=jax.ShapeDtypeStruct((NC * ACC_R,), jnp.float32),
        scratch_types=[
            pltpu.VMEM((CPW, CHUNK), jnp.int32),
            pltpu.VMEM((CHUNK,), jnp.float32),
            pltpu.VMEM_SHARED((ACC_R,), jnp.float32),
            pltpu.SemaphoreType.DMA,
        ],
    )
    def hist_kernel(dst_hbm, ones_hbm, zeros_hbm, out_hbm,
                    idx_dst_v, ones_v, acc_sh, sem):
        c = lax.axis_index("c")
        s = lax.axis_index("s")
        wid = s * NC + c
        pltpu.sync_copy(zeros_hbm.at[pl.ds(s * RPT, RPT)],
                        acc_sh.at[pl.ds(s * RPT, RPT)])
        pltpu.sync_copy(dst_hbm.at[wid], idx_dst_v)
        pltpu.sync_copy(ones_hbm, ones_v)
        plsc.subcore_barrier()

        @pl.loop(0, CPW, step=WAVE)
        def _(j0):
            for b in range(WAVE):
                pltpu.async_copy(ones_v, acc_sh.at[idx_dst_v.at[j0 + b]],
                                 sem, add=True)
            for b in range(WAVE):
                pltpu.make_async_copy(ones_v, acc_sh.at[idx_dst_v.at[j0 + b]],
                                      sem).wait()

        plsc.subcore_barrier()
        pltpu.sync_copy(acc_sh.at[pl.ds(s * RPT, RPT)],
                        out_hbm.at[pl.ds(c * ACC_R + s * RPT, RPT)])

    return hist_kernel


def _make_scalar_scatter(NW, NC, NS, CPW, ACC_R, N):
    RPT = ACC_R // NS
    mesh = plsc.VectorSubcoreMesh(core_axis_name="c", subcore_axis_name="s")

    @functools.partial(
        pl.kernel,
        mesh=mesh,
        compiler_params=pltpu.CompilerParams(use_tc_tiling_on_sc=False,
                                             needs_layout_passes=False),
        out_type=jax.ShapeDtypeStruct((NC * ACC_R,), jnp.float32),
        scratch_types=[
            pltpu.VMEM((CPW, CHUNK), jnp.int32),
            pltpu.VMEM((CPW, CHUNK), jnp.int32),
            pltpu.VMEM((N,), jnp.float32),
            pltpu.VMEM((WAVE, CHUNK), jnp.float32),
            pltpu.VMEM_SHARED((ACC_R,), jnp.float32),
            pltpu.SemaphoreType.DMA,
        ],
    )
    def scatter_kernel(src_hbm, dst_hbm, vals_hbm, zeros_hbm, out_hbm,
                       idx_src_v, idx_dst_v, g2_v, vals_v, acc_sh, sem):
        c = lax.axis_index("c")
        s = lax.axis_index("s")
        wid = s * NC + c
        pltpu.sync_copy(zeros_hbm.at[pl.ds(s * RPT, RPT)],
                        acc_sh.at[pl.ds(s * RPT, RPT)])
        pltpu.sync_copy(src_hbm.at[wid], idx_src_v)
        pltpu.sync_copy(dst_hbm.at[wid], idx_dst_v)
        pltpu.sync_copy(vals_hbm, g2_v)
        plsc.subcore_barrier()

        @pl.loop(0, CPW, step=WAVE)
        def _(j0):
            for b in range(WAVE):
                j = j0 + b
                for k in range(CHUNK // 16):
                    iv = idx_src_v[j, pl.ds(k * 16, 16)]
                    vals_v[b, pl.ds(k * 16, 16)] = plsc.load_gather(g2_v, [iv])
                pltpu.async_copy(vals_v.at[b], acc_sh.at[idx_dst_v.at[j]],
                                 sem, add=True)
            for b in range(WAVE):
                pltpu.make_async_copy(vals_v.at[b],
                                      acc_sh.at[idx_dst_v.at[j0 + b]],
                                      sem).wait()

        plsc.subcore_barrier()
        pltpu.sync_copy(acc_sh.at[pl.ds(s * RPT, RPT)],
                        out_hbm.at[pl.ds(c * ACC_R + s * RPT, RPT)])

    return scatter_kernel


def _make_row_scatter(NW, NC, NS, CPW, ACC_R, H):
    RPT = ACC_R // NS
    mesh = plsc.VectorSubcoreMesh(core_axis_name="c", subcore_axis_name="s")

    @functools.partial(
        pl.kernel,
        mesh=mesh,
        compiler_params=pltpu.CompilerParams(use_tc_tiling_on_sc=False,
                                             needs_layout_passes=False),
        out_type=jax.ShapeDtypeStruct((NC, ACC_R, H), jnp.float32),
        scratch_types=[
            pltpu.VMEM((CPW, CHUNK), jnp.int32),
            pltpu.VMEM((CPW, CHUNK), jnp.int32),
            pltpu.VMEM((2, RWAVE, CHUNK, H), jnp.float32),
            pltpu.VMEM_SHARED((ACC_R, H), jnp.float32),
            pltpu.SemaphoreType.DMA,
            pltpu.SemaphoreType.DMA,
        ],
    )
    def row_scatter_kernel(src_hbm, dst_hbm, g1_hbm, zeros_hbm, out_hbm,
                           idx_src_v, idx_dst_v, rows_v, acc_sh, gsem, ssem):
        c = lax.axis_index("c")
        s = lax.axis_index("s")
        wid = s * NC + c
        pltpu.sync_copy(zeros_hbm.at[pl.ds(s * RPT, RPT)],
                        acc_sh.at[pl.ds(s * RPT, RPT)])
        pltpu.sync_copy(src_hbm.at[wid], idx_src_v)
        pltpu.sync_copy(dst_hbm.at[wid], idx_dst_v)
        plsc.subcore_barrier()

        NWAVES = CPW // RWAVE

        for b in range(RWAVE):
            pltpu.async_copy(g1_hbm.at[idx_src_v.at[b]],
                             rows_v.at[0].at[b], gsem)

        @pl.loop(0, NWAVES, step=2)
        def _(w0):
            for st in range(2):
                w = w0 + st
                for b in range(RWAVE):
                    pltpu.make_async_copy(
                        g1_hbm.at[idx_src_v.at[w * RWAVE + b]],
                        rows_v.at[st].at[b], gsem).wait()
                for b in range(RWAVE):
                    pltpu.async_copy(
                        rows_v.at[st].at[b],
                        acc_sh.at[idx_dst_v.at[w * RWAVE + b]],
                        ssem, add=True)

                @pl.when(w >= 1)
                def _drain_prev():
                    for b in range(RWAVE):
                        pltpu.make_async_copy(
                            rows_v.at[1 - st].at[b],
                            acc_sh.at[idx_dst_v.at[(w - 1) * RWAVE + b]],
                            ssem).wait()

                @pl.when(w + 1 < NWAVES)
                def _fire_next():
                    for b in range(RWAVE):
                        pltpu.async_copy(
                            g1_hbm.at[idx_src_v.at[(w + 1) * RWAVE + b]],
                            rows_v.at[1 - st].at[b], gsem)

        for b in range(RWAVE):
            pltpu.make_async_copy(
                rows_v.at[1].at[b],
                acc_sh.at[idx_dst_v.at[(NWAVES - 1) * RWAVE + b]],
                ssem).wait()

        plsc.subcore_barrier()
        pltpu.sync_copy(acc_sh.at[pl.ds(s * RPT, RPT)],
                        out_hbm.at[c].at[pl.ds(s * RPT, RPT)])

    return row_scatter_kernel


def _tc1_body(x_ref, w1_ref, c0_ref, c1_ref, g1_ref, dis_ref):
    deg = 1.0 + c0_ref[...] + c1_ref[...]
    dis = lax.rsqrt(deg)
    h1 = jnp.dot(x_ref[...], w1_ref[...], preferred_element_type=jnp.float32)
    g1_ref[...] = h1 * dis
    dis_ref[...] = dis


def _tc2_body(dis_ref, g1_ref, p0_ref, p1_ref, b1_ref, w2_ref, g2_ref):
    dis = dis_ref[...]
    out1 = dis * (p0_ref[...] + p1_ref[...] + g1_ref[...]) + b1_ref[...]
    z = jnp.maximum(out1, 0.0)
    h2 = jnp.dot(z, w2_ref[...], preferred_element_type=jnp.float32)
    g2_ref[...] = dis * h2


def _tc3_body(dis_ref, g2_ref, q0_ref, q1_ref, b2_ref, out_ref):
    out_ref[...] = (dis_ref[...] * (q0_ref[...] + q1_ref[...] + g2_ref[...])
                    + b2_ref[...])


def kernel(x, edge_index, W1, b1, W2, b2):
    N, D = x.shape
    H = W1.shape[1]
    O = W2.shape[1]
    E = edge_index.shape[1]

    info = plsc.get_sparse_core_info()
    NC, NS = info.num_cores, info.num_subcores
    NW = NC * NS
    CPW = -(-E // (NW * CHUNK))
    CPW = ((CPW + 2 * WAVE - 1) // (2 * WAVE)) * (2 * WAVE)
    EPAD = NW * CPW * CHUNK
    ACC_R = ((N + NS * 128 - 1) // (NS * 128)) * (NS * 128)
    if ACC_R == N:
        ACC_R += NS * 128
    pad = EPAD - E

    src = edge_index[0]
    dst = edge_index[1]
    pad_i = jnp.arange(pad, dtype=jnp.int32)
    srcp = jnp.concatenate([src, pad_i % N]).reshape(NW, CPW, CHUNK)
    dstp = jnp.concatenate([dst, N + pad_i % (ACC_R - N)]).reshape(NW, CPW, CHUNK)

    zeros2 = jnp.zeros((ACC_R, H), jnp.float32)
    zeros1 = jnp.zeros((ACC_R,), jnp.float32)
    ones_c = jnp.ones((CHUNK,), jnp.float32)

    hist = _make_hist(NW, NC, NS, CPW, ACC_R)
    scalar_scatter = _make_scalar_scatter(NW, NC, NS, CPW, ACC_R, N)
    row_scatter = _make_row_scatter(NW, NC, NS, CPW, ACC_R, H)

    cnt = hist(dstp, ones_c, zeros1).reshape(NC, ACC_R)
    c0 = cnt[0, :N, None]
    c1 = cnt[1, :N, None]

    BN = 2000
    grid = (N // BN,)
    g1, dis = pl.pallas_call(
        _tc1_body,
        grid=grid,
        in_specs=[
            pl.BlockSpec((BN, D), lambda i: (i, 0)),
            pl.BlockSpec((D, H), lambda i: (0, 0)),
            pl.BlockSpec((BN, 1), lambda i: (i, 0)),
            pl.BlockSpec((BN, 1), lambda i: (i, 0)),
        ],
        out_specs=[
            pl.BlockSpec((BN, H), lambda i: (i, 0)),
            pl.BlockSpec((BN, 1), lambda i: (i, 0)),
        ],
        out_shape=[
            jax.ShapeDtypeStruct((N, H), jnp.float32),
            jax.ShapeDtypeStruct((N, 1), jnp.float32),
        ],
    )(x, W1, c0, c1)

    p = row_scatter(srcp, dstp, g1, zeros2)
    p0 = p[0, :N, :]
    p1 = p[1, :N, :]

    g2 = pl.pallas_call(
        _tc2_body,
        grid=grid,
        in_specs=[
            pl.BlockSpec((BN, 1), lambda i: (i, 0)),
            pl.BlockSpec((BN, H), lambda i: (i, 0)),
            pl.BlockSpec((BN, H), lambda i: (i, 0)),
            pl.BlockSpec((BN, H), lambda i: (i, 0)),
            pl.BlockSpec((1, H), lambda i: (0, 0)),
            pl.BlockSpec((H, O), lambda i: (0, 0)),
        ],
        out_specs=pl.BlockSpec((BN, O), lambda i: (i, 0)),
        out_shape=jax.ShapeDtypeStruct((N, O), jnp.float32),
    )(dis, g1, p0, p1, b1[None, :], W2)

    q = scalar_scatter(srcp, dstp, g2[:, 0], zeros1).reshape(NC, ACC_R)
    q0 = q[0, :N, None]
    q1 = q[1, :N, None]

    out = pl.pallas_call(
        _tc3_body,
        grid=grid,
        in_specs=[
            pl.BlockSpec((BN, 1), lambda i: (i, 0)),
            pl.BlockSpec((BN, O), lambda i: (i, 0)),
            pl.BlockSpec((BN, 1), lambda i: (i, 0)),
            pl.BlockSpec((BN, 1), lambda i: (i, 0)),
            pl.BlockSpec((1, 1), lambda i: (0, 0)),
        ],
        out_specs=pl.BlockSpec((BN, O), lambda i: (i, 0)),
        out_shape=jax.ShapeDtypeStruct((N, O), jnp.float32),
    )(dis, g2, q0, q1, b2[None, :])

    return out


# speedup vs baseline: 65.7658x; 1.0400x over previous
import functools

import jax
import jax.numpy as jnp
from jax import lax
from jax.experimental import pallas as pl
from jax.experimental.pallas import tpu as pltpu
from jax.experimental.pallas import tpu_sc as plsc


CHUNK = 128
WAVE = 8
RWAVE = 8


def _make_hist(NW, NC, NS, CPW, ACC_R):
    RPT = ACC_R // NS
    mesh = plsc.VectorSubcoreMesh(core_axis_name="c", subcore_axis_name="s")

    @functools.partial(
        pl.kernel,
        mesh=mesh,
        compiler_params=pltpu.CompilerParams(use_tc_tiling_on_sc=False,
                                             needs_layout_passes=False),
        out_type=jax.ShapeDtypeStruct((NC * ACC_R,), jnp.float32),
        scratch_types=[
            pltpu.VMEM((CPW, CHUNK), jnp.int32),
            pltpu.VMEM((CHUNK,), jnp.float32),
            pltpu.VMEM_SHARED((ACC_R,), jnp.float32),
            pltpu.SemaphoreType.DMA,
        ],
    )
    def hist_kernel(dst_hbm, ones_hbm, zeros_hbm, out_hbm,
                    idx_dst_v, ones_v, acc_sh, sem):
        c = lax.axis_index("c")
        s = lax.axis_index("s")
        wid = s * NC + c
        pltpu.sync_copy(zeros_hbm.at[pl.ds(s * RPT, RPT)],
                        acc_sh.at[pl.ds(s * RPT, RPT)])
        pltpu.sync_copy(dst_hbm.at[wid], idx_dst_v)
        pltpu.sync_copy(ones_hbm, ones_v)
        plsc.subcore_barrier()

        @pl.loop(0, CPW, step=WAVE)
        def _(j0):
            for b in range(WAVE):
                pltpu.async_copy(ones_v, acc_sh.at[idx_dst_v.at[j0 + b]],
                                 sem, add=True)
            for b in range(WAVE):
                pltpu.make_async_copy(ones_v, acc_sh.at[idx_dst_v.at[j0 + b]],
                                      sem).wait()

        plsc.subcore_barrier()
        pltpu.sync_copy(acc_sh.at[pl.ds(s * RPT, RPT)],
                        out_hbm.at[pl.ds(c * ACC_R + s * RPT, RPT)])

    return hist_kernel


def _make_scalar_scatter(NW, NC, NS, CPW, ACC_R, N):
    RPT = ACC_R // NS
    mesh = plsc.VectorSubcoreMesh(core_axis_name="c", subcore_axis_name="s")

    @functools.partial(
        pl.kernel,
        mesh=mesh,
        compiler_params=pltpu.CompilerParams(use_tc_tiling_on_sc=False,
                                             needs_layout_passes=False),
        out_type=jax.ShapeDtypeStruct((NC * ACC_R,), jnp.float32),
        scratch_types=[
            pltpu.VMEM((CPW, CHUNK), jnp.int32),
            pltpu.VMEM((CPW, CHUNK), jnp.int32),
            pltpu.VMEM((N,), jnp.float32),
            pltpu.VMEM((WAVE, CHUNK), jnp.float32),
            pltpu.VMEM_SHARED((ACC_R,), jnp.float32),
            pltpu.SemaphoreType.DMA,
        ],
    )
    def scatter_kernel(src_hbm, dst_hbm, vals_hbm, zeros_hbm, out_hbm,
                       idx_src_v, idx_dst_v, g2_v, vals_v, acc_sh, sem):
        c = lax.axis_index("c")
        s = lax.axis_index("s")
        wid = s * NC + c
        pltpu.sync_copy(zeros_hbm.at[pl.ds(s * RPT, RPT)],
                        acc_sh.at[pl.ds(s * RPT, RPT)])
        pltpu.sync_copy(src_hbm.at[wid], idx_src_v)
        pltpu.sync_copy(dst_hbm.at[wid], idx_dst_v)
        pltpu.sync_copy(vals_hbm, g2_v)
        plsc.subcore_barrier()

        @pl.loop(0, CPW, step=WAVE)
        def _(j0):
            for b in range(WAVE):
                j = j0 + b
                for k in range(CHUNK // 16):
                    iv = idx_src_v[j, pl.ds(k * 16, 16)]
                    vals_v[b, pl.ds(k * 16, 16)] = plsc.load_gather(g2_v, [iv])
                pltpu.async_copy(vals_v.at[b], acc_sh.at[idx_dst_v.at[j]],
                                 sem, add=True)
            for b in range(WAVE):
                pltpu.make_async_copy(vals_v.at[b],
                                      acc_sh.at[idx_dst_v.at[j0 + b]],
                                      sem).wait()

        plsc.subcore_barrier()
        pltpu.sync_copy(acc_sh.at[pl.ds(s * RPT, RPT)],
                        out_hbm.at[pl.ds(c * ACC_R + s * RPT, RPT)])

    return scatter_kernel


def _make_row_scatter(NW, NC, NS, CPW, ACC_R, H):
    RPT = ACC_R // NS
    mesh = plsc.VectorSubcoreMesh(core_axis_name="c", subcore_axis_name="s")

    @functools.partial(
        pl.kernel,
        mesh=mesh,
        compiler_params=pltpu.CompilerParams(use_tc_tiling_on_sc=False,
                                             needs_layout_passes=False),
        out_type=jax.ShapeDtypeStruct((NC, ACC_R, H), jnp.float32),
        scratch_types=[
            pltpu.VMEM((CPW, CHUNK), jnp.int32),
            pltpu.VMEM((CPW, CHUNK), jnp.int32),
            pltpu.VMEM((2, RWAVE, CHUNK, H), jnp.float32),
            pltpu.VMEM_SHARED((ACC_R, H), jnp.float32),
            pltpu.SemaphoreType.DMA,
            pltpu.SemaphoreType.DMA,
        ],
    )
    def row_scatter_kernel(src_hbm, dst_hbm, g1_hbm, zeros_hbm, out_hbm,
                           idx_src_v, idx_dst_v, rows_v, acc_sh, gsem, ssem):
        c = lax.axis_index("c")
        s = lax.axis_index("s")
        wid = s * NC + c
        pltpu.sync_copy(zeros_hbm.at[pl.ds(s * RPT, RPT)],
                        acc_sh.at[pl.ds(s * RPT, RPT)])
        pltpu.sync_copy(src_hbm.at[wid], idx_src_v)
        pltpu.sync_copy(dst_hbm.at[wid], idx_dst_v)
        plsc.subcore_barrier()

        NWAVES = CPW // RWAVE

        for b in range(RWAVE):
            pltpu.async_copy(g1_hbm.at[idx_src_v.at[b]],
                             rows_v.at[0].at[b], gsem)

        @pl.loop(0, NWAVES, step=2)
        def _(w0):
            for st in range(2):
                w = w0 + st
                for b in range(RWAVE):
                    pltpu.make_async_copy(
                        g1_hbm.at[idx_src_v.at[w * RWAVE + b]],
                        rows_v.at[st].at[b], gsem).wait()
                for b in range(RWAVE):
                    pltpu.async_copy(
                        rows_v.at[st].at[b],
                        acc_sh.at[idx_dst_v.at[w * RWAVE + b]],
                        ssem, add=True)

                @pl.when(w >= 1)
                def _drain_prev():
                    for b in range(RWAVE):
                        pltpu.make_async_copy(
                            rows_v.at[1 - st].at[b],
                            acc_sh.at[idx_dst_v.at[(w - 1) * RWAVE + b]],
                            ssem).wait()

                @pl.when(w + 1 < NWAVES)
                def _fire_next():
                    for b in range(RWAVE):
                        pltpu.async_copy(
                            g1_hbm.at[idx_src_v.at[(w + 1) * RWAVE + b]],
                            rows_v.at[1 - st].at[b], gsem)

        for b in range(RWAVE):
            pltpu.make_async_copy(
                rows_v.at[1].at[b],
                acc_sh.at[idx_dst_v.at[(NWAVES - 1) * RWAVE + b]],
                ssem).wait()

        plsc.subcore_barrier()
        pltpu.sync_copy(acc_sh.at[pl.ds(s * RPT, RPT)],
                        out_hbm.at[c].at[pl.ds(s * RPT, RPT)])

    return row_scatter_kernel


def _tc1_body(x_ref, w1_ref, c0_ref, c1_ref, g1_ref, dis_ref):
    deg = 1.0 + c0_ref[...] + c1_ref[...]
    dis = lax.rsqrt(deg)
    h1 = jnp.dot(x_ref[...], w1_ref[...], preferred_element_type=jnp.float32)
    g1_ref[...] = h1 * dis
    dis_ref[...] = dis


def _tc2_body(dis_ref, g1_ref, p0_ref, p1_ref, b1_ref, w2_ref, g2_ref):
    dis = dis_ref[...]
    out1 = dis * (p0_ref[...] + p1_ref[...] + g1_ref[...]) + b1_ref[...]
    z = jnp.maximum(out1, 0.0)
    h2 = jnp.dot(z, w2_ref[...], preferred_element_type=jnp.float32)
    g2_ref[...] = dis * h2


def _tc3_body(dis_ref, g2_ref, q0_ref, q1_ref, b2_ref, out_ref):
    out_ref[...] = (dis_ref[...] * (q0_ref[...] + q1_ref[...] + g2_ref[...])
                    + b2_ref[...])


def kernel(x, edge_index, W1, b1, W2, b2):
    N, D = x.shape
    H = W1.shape[1]
    O = W2.shape[1]
    E = edge_index.shape[1]

    info = plsc.get_sparse_core_info()
    NC, NS = info.num_cores, info.num_subcores
    NW = NC * NS
    CPW = -(-E // (NW * CHUNK))
    CPW = ((CPW + 2 * WAVE - 1) // (2 * WAVE)) * (2 * WAVE)
    EPAD = NW * CPW * CHUNK
    ACC_R = ((N + NS * 128 - 1) // (NS * 128)) * (NS * 128)
    if ACC_R == N:
        ACC_R += NS * 128
    pad = EPAD - E

    src = edge_index[0]
    dst = edge_index[1]
    pad_i = jnp.arange(pad, dtype=jnp.int32)
    srcp = jnp.concatenate([src, pad_i % N]).reshape(NW, CPW, CHUNK)
    dstp = jnp.concatenate([dst, N + pad_i % (ACC_R - N)]).reshape(NW, CPW, CHUNK)

    zeros2 = jnp.zeros((ACC_R, H), jnp.float32)
    zeros1 = jnp.zeros((ACC_R,), jnp.float32)
    ones_c = jnp.ones((CHUNK,), jnp.float32)

    hist = _make_hist(NW, NC, NS, CPW, ACC_R)
    scalar_scatter = _make_scalar_scatter(NW, NC, NS, CPW, ACC_R, N)
    row_scatter = _make_row_scatter(NW, NC, NS, CPW, ACC_R, H)

    cnt = hist(dstp, ones_c, zeros1).reshape(NC, ACC_R)
    c0 = cnt[0, :N, None]
    c1 = cnt[1, :N, None]

    BN = 2000
    grid = (N // BN,)
    g1, dis = pl.pallas_call(
        _tc1_body,
        grid=grid,
        in_specs=[
            pl.BlockSpec((BN, D), lambda i: (i, 0)),
            pl.BlockSpec((D, H), lambda i: (0, 0)),
            pl.BlockSpec((BN, 1), lambda i: (i, 0)),
            pl.BlockSpec((BN, 1), lambda i: (i, 0)),
        ],
        out_specs=[
            pl.BlockSpec((BN, H), lambda i: (i, 0)),
            pl.BlockSpec((BN, 1), lambda i: (i, 0)),
        ],
        out_shape=[
            jax.ShapeDtypeStruct((N, H), jnp.float32),
            jax.ShapeDtypeStruct((N, 1), jnp.float32),
        ],
    )(x, W1, c0, c1)

    p = row_scatter(srcp, dstp, g1, zeros2)
    p0 = p[0, :N, :]
    p1 = p[1, :N, :]

    g2 = pl.pallas_call(
        _tc2_body,
        grid=grid,
        in_specs=[
            pl.BlockSpec((BN, 1), lambda i: (i, 0)),
            pl.BlockSpec((BN, H), lambda i: (i, 0)),
            pl.BlockSpec((BN, H), lambda i: (i, 0)),
            pl.BlockSpec((BN, H), lambda i: (i, 0)),
            pl.BlockSpec((1, H), lambda i: (0, 0)),
            pl.BlockSpec((H, O), lambda i: (0, 0)),
        ],
        out_specs=pl.BlockSpec((BN, O), lambda i: (i, 0)),
        out_shape=jax.ShapeDtypeStruct((N, O), jnp.float32),
    )(dis, g1, p0, p1, b1[None, :], W2)

    q = scalar_scatter(srcp, dstp, g2[:, 0], zeros1).reshape(NC, ACC_R)
    q0 = q[0, :N, None]
    q1 = q[1, :N, None]

    out = pl.pallas_call(
        _tc3_body,
        grid=grid,
        in_specs=[
            pl.BlockSpec((BN, 1), lambda i: (i, 0)),
            pl.BlockSpec((BN, O), lambda i: (i, 0)),
            pl.BlockSpec((BN, 1), lambda i: (i, 0)),
            pl.BlockSpec((BN, 1), lambda i: (i, 0)),
            pl.BlockSpec((1, 1), lambda i: (0, 0)),
        ],
        out_specs=pl.BlockSpec((BN, O), lambda i: (i, 0)),
        out_shape=jax.ShapeDtypeStruct((N, O), jnp.float32),
    )(dis, g2, q0, q1, b2[None, :])

    return out


# speedup vs baseline: 87.9748x; 1.3377x over previous
import functools

import jax
import jax.numpy as jnp
from jax import lax
from jax.experimental import pallas as pl
from jax.experimental.pallas import tpu as pltpu
from jax.experimental.pallas import tpu_sc as plsc


CHUNK = 128
WAVE = 8
RWAVE = 8


def _bf16_round(v):
    u = plsc.bitcast(v, jnp.int32)
    r = (u + 0x7FFF + ((u >> 16) & 1)) & jnp.int32(-65536)
    return plsc.bitcast(r, jnp.float32)


def _newton_rsqrt(x):
    xi = plsc.bitcast(x, jnp.int32)
    y = plsc.bitcast(jnp.int32(0x5F3759DF) - (xi >> 1), jnp.float32)
    xh = 0.5 * x
    for _ in range(3):
        y = y * (1.5 - xh * y * y)
    return y


def _make_sc_all(NC, NS, CPW, ACC_R, H, N):
    NW = NC * NS
    RPT = ACC_R // NS
    OPT2 = ACC_R // NW
    HALF = ACC_R // NC
    NWAVES = CPW // RWAVE
    mesh = plsc.VectorSubcoreMesh(core_axis_name="c", subcore_axis_name="s")

    @functools.partial(
        pl.kernel,
        mesh=mesh,
        compiler_params=pltpu.CompilerParams(use_tc_tiling_on_sc=False,
                                             needs_layout_passes=False),
        out_type=[
            jax.ShapeDtypeStruct((ACC_R,), jnp.float32),
            jax.ShapeDtypeStruct((NC * ACC_R,), jnp.float32),
            jax.ShapeDtypeStruct((ACC_R, H), jnp.float32),
            jax.ShapeDtypeStruct((NC, ACC_R, H), jnp.float32),
            jax.ShapeDtypeStruct((NC * ACC_R,), jnp.float32),
        ],
        scratch_types=[
            pltpu.VMEM((CPW, CHUNK), jnp.int32),
            pltpu.VMEM((CPW, CHUNK), jnp.int32),
            pltpu.VMEM((2, RWAVE, CHUNK, H), jnp.float32),
            pltpu.VMEM((RPT, H), jnp.float32),
            pltpu.VMEM((RPT, H), jnp.float32),
            pltpu.VMEM((RPT, H), jnp.float32),
            pltpu.VMEM((RPT,), jnp.float32),
            pltpu.VMEM((RPT,), jnp.float32),
            pltpu.VMEM((RPT,), jnp.float32),
            pltpu.VMEM((ACC_R,), jnp.float32),
            pltpu.VMEM((WAVE, CHUNK), jnp.float32),
            pltpu.VMEM((48,), jnp.float32),
            pltpu.VMEM_SHARED((ACC_R,), jnp.float32),
            pltpu.VMEM_SHARED((ACC_R, H), jnp.float32),
            pltpu.VMEM_SHARED((ACC_R,), jnp.float32),
            pltpu.VMEM_SHARED((ACC_R,), jnp.float32),
            pltpu.VMEM_SHARED((ACC_R,), jnp.float32),
            pltpu.SemaphoreType.DMA,
            pltpu.SemaphoreType.DMA,
            pltpu.SemaphoreType.REGULAR,
        ],
    )
    def sc_all(src_hbm, dst_hbm, h1_hbm, cst_hbm, zeros1_hbm, zeros2_hbm,
               out_hbm, xcnt_hbm, xg1_hbm, xp_hbm, xq_hbm,
               idx_src_v, idx_dst_v, rows_v, slab_a, slab_b, slab_c,
               vec_a, vec_b, vec_c, g2_v, vals_v, cst_v,
               cnt_sh, acc1_sh, dis_sh, g2_sh, acc2_sh,
               gsem, ssem, xsem):
        c = lax.axis_index("c")
        s = lax.axis_index("s")
        wid = s * NC + c
        wid2 = c * NS + s

        def xbar():
            plsc.subcore_barrier()

            @pl.when(s == 0)
            def _():
                pltpu.semaphore_signal(xsem, 1, core_index=1 - c)
                pltpu.semaphore_wait(xsem, 1)

            plsc.subcore_barrier()

        pltpu.sync_copy(zeros1_hbm.at[pl.ds(s * RPT, RPT)],
                        cnt_sh.at[pl.ds(s * RPT, RPT)])
        pltpu.sync_copy(zeros1_hbm.at[pl.ds(s * RPT, RPT)],
                        acc2_sh.at[pl.ds(s * RPT, RPT)])
        pltpu.sync_copy(zeros2_hbm.at[pl.ds(s * RPT, RPT)],
                        acc1_sh.at[pl.ds(s * RPT, RPT)])
        pltpu.sync_copy(src_hbm.at[wid], idx_src_v)
        pltpu.sync_copy(dst_hbm.at[wid], idx_dst_v)
        pltpu.sync_copy(cst_hbm, cst_v)
        for k in range(CHUNK // 16):
            vals_v[0, pl.ds(k * 16, 16)] = jnp.ones((16,), jnp.float32)
        plsc.subcore_barrier()

        @pl.loop(0, CPW, step=WAVE)
        def _(j0):
            for b in range(WAVE):
                pltpu.async_copy(vals_v.at[0],
                                 cnt_sh.at[idx_dst_v.at[j0 + b]],
                                 ssem, add=True)
            for b in range(WAVE):
                pltpu.make_async_copy(vals_v.at[0],
                                      cnt_sh.at[idx_dst_v.at[j0 + b]],
                                      ssem).wait()

        plsc.subcore_barrier()

        pltpu.sync_copy(cnt_sh.at[pl.ds(s * RPT, RPT)],
                        xcnt_hbm.at[pl.ds(c * ACC_R + s * RPT, RPT)])
        xbar()

        pltpu.sync_copy(xcnt_hbm.at[pl.ds((1 - c) * ACC_R + s * RPT, RPT)],
                        vec_b)
        pltpu.sync_copy(cnt_sh.at[pl.ds(s * RPT, RPT)], vec_a)
        for k in range(RPT // 16):
            deg = 1.0 + vec_a[pl.ds(k * 16, 16)] + vec_b[pl.ds(k * 16, 16)]
            vec_a[pl.ds(k * 16, 16)] = _newton_rsqrt(deg)
        pltpu.sync_copy(vec_a, dis_sh.at[pl.ds(s * RPT, RPT)])
        plsc.subcore_barrier()

        pltpu.sync_copy(h1_hbm.at[pl.ds(wid2 * OPT2, OPT2)],
                        slab_a.at[pl.ds(0, OPT2)])
        pltpu.sync_copy(dis_sh.at[pl.ds(wid2 * OPT2, OPT2)],
                        vec_b.at[pl.ds(0, OPT2)])

        @pl.loop(0, OPT2)
        def _(r):
            dsp = plsc.load_gather(vec_b, [jnp.zeros((16,), jnp.int32) + r])
            slab_a[r, :] = slab_a[r, :] * dsp

        pltpu.sync_copy(slab_a.at[pl.ds(0, OPT2)],
                        xg1_hbm.at[pl.ds(wid2 * OPT2, OPT2)])
        xbar()

        for b in range(RWAVE):
            pltpu.async_copy(xg1_hbm.at[idx_src_v.at[b]],
                             rows_v.at[0].at[b], gsem)

        @pl.loop(0, NWAVES, step=2)
        def _(w0):
            for st in range(2):
                w = w0 + st
                for b in range(RWAVE):
                    pltpu.make_async_copy(
                        xg1_hbm.at[idx_src_v.at[w * RWAVE + b]],
                        rows_v.at[st].at[b], gsem).wait()
                for b in range(RWAVE):
                    pltpu.async_copy(
                        rows_v.at[st].at[b],
                        acc1_sh.at[idx_dst_v.at[w * RWAVE + b]],
                        ssem, add=True)

                @pl.when(w >= 1)
                def _drain_prev():
                    for b in range(RWAVE):
                        pltpu.make_async_copy(
                            rows_v.at[1 - st].at[b],
                            acc1_sh.at[idx_dst_v.at[(w - 1) * RWAVE + b]],
                            ssem).wait()

                @pl.when(w + 1 < NWAVES)
                def _fire_next():
                    for b in range(RWAVE):
                        pltpu.async_copy(
                            xg1_hbm.at[idx_src_v.at[(w + 1) * RWAVE + b]],
                            rows_v.at[1 - st].at[b], gsem)

        for b in range(RWAVE):
            pltpu.make_async_copy(
                rows_v.at[1].at[b],
                acc1_sh.at[idx_dst_v.at[(NWAVES - 1) * RWAVE + b]],
                ssem).wait()

        plsc.subcore_barrier()
        pltpu.sync_copy(acc1_sh.at[pl.ds(s * RPT, RPT)],
                        xp_hbm.at[c].at[pl.ds(s * RPT, RPT)])
        xbar()

        b1v = cst_v[pl.ds(0, 16)]
        w2v = cst_v[pl.ds(16, 16)]
        b2v = cst_v[pl.ds(32, 16)]
        pltpu.sync_copy(acc1_sh.at[pl.ds(s * RPT, RPT)], slab_a)
        pltpu.sync_copy(xp_hbm.at[1 - c].at[pl.ds(s * RPT, RPT)], slab_b)
        pltpu.sync_copy(xg1_hbm.at[pl.ds(s * RPT, RPT)], slab_c)
        pltpu.sync_copy(dis_sh.at[pl.ds(s * RPT, RPT)], vec_a)

        w2b = _bf16_round(w2v)

        @pl.loop(0, RPT)
        def _(r):
            rv = jnp.zeros((16,), jnp.int32) + r
            t = slab_a[r, :] + slab_b[r, :] + slab_c[r, :]
            dsp = plsc.load_gather(vec_a, [rv])
            z = jnp.maximum(dsp * t + b1v, 0.0)
            tot = jnp.sum(_bf16_round(z) * w2b)
            plsc.store_scatter(vec_b, [rv],
                               (jnp.zeros((16,), jnp.float32) + tot) * dsp)

        pltpu.sync_copy(vec_b, g2_sh.at[pl.ds(s * RPT, RPT)])
        plsc.subcore_barrier()
        pltpu.sync_copy(g2_sh, g2_v)

        @pl.loop(0, CPW, step=WAVE)
        def _(j0):
            for b in range(WAVE):
                j = j0 + b
                for k in range(CHUNK // 16):
                    iv = idx_src_v[j, pl.ds(k * 16, 16)]
                    vals_v[b, pl.ds(k * 16, 16)] = plsc.load_gather(g2_v, [iv])
                pltpu.async_copy(vals_v.at[b], acc2_sh.at[idx_dst_v.at[j]],
                                 ssem, add=True)
            for b in range(WAVE):
                pltpu.make_async_copy(vals_v.at[b],
                                      acc2_sh.at[idx_dst_v.at[j0 + b]],
                                      ssem).wait()

        plsc.subcore_barrier()
        pltpu.sync_copy(acc2_sh.at[pl.ds(s * RPT, RPT)],
                        xq_hbm.at[pl.ds(c * ACC_R + s * RPT, RPT)])
        xbar()

        base = c * HALF + s * OPT2
        pltpu.sync_copy(acc2_sh.at[pl.ds(base, OPT2)], vec_a.at[pl.ds(0, OPT2)])
        pltpu.sync_copy(xq_hbm.at[pl.ds((1 - c) * ACC_R + base, OPT2)],
                        vec_b.at[pl.ds(0, OPT2)])
        pltpu.sync_copy(dis_sh.at[pl.ds(base, OPT2)], vec_c.at[pl.ds(0, OPT2)])
        for k in range(OPT2 // 16):
            q = vec_a[pl.ds(k * 16, 16)] + vec_b[pl.ds(k * 16, 16)]
            g = g2_v[pl.ds(base + k * 16, 16)]
            d = vec_c[pl.ds(k * 16, 16)]
            vec_a[pl.ds(k * 16, 16)] = d * (q + g) + b2v
        pltpu.sync_copy(vec_a.at[pl.ds(0, OPT2)],
                        out_hbm.at[pl.ds(base, OPT2)])

    return sc_all


def _tc0_body(x_ref, w1_ref, h1_ref):
    h1_ref[...] = jnp.dot(x_ref[...], w1_ref[...],
                          preferred_element_type=jnp.float32)


def kernel(x, edge_index, W1, b1, W2, b2):
    N, D = x.shape
    H = W1.shape[1]
    E = edge_index.shape[1]

    info = plsc.get_sparse_core_info()
    NC, NS = info.num_cores, info.num_subcores
    NW = NC * NS
    CPW = -(-E // (NW * CHUNK))
    CPW = ((CPW + 2 * WAVE - 1) // (2 * WAVE)) * (2 * WAVE)
    EPAD = NW * CPW * CHUNK
    ACC_R = ((N + NS * 128 - 1) // (NS * 128)) * (NS * 128)
    if ACC_R == N:
        ACC_R += NS * 128
    pad = EPAD - E

    src = edge_index[0]
    dst = edge_index[1]
    pad_i = jnp.arange(pad, dtype=jnp.int32)
    srcp = jnp.concatenate([src, pad_i % N]).reshape(NW, CPW, CHUNK)
    dstp = jnp.concatenate([dst, N + pad_i % (ACC_R - N)]).reshape(NW, CPW, CHUNK)

    zeros2 = jnp.zeros((ACC_R, H), jnp.float32)
    zeros1 = jnp.zeros((ACC_R,), jnp.float32)
    consts = jnp.concatenate([b1, W2[:, 0], jnp.broadcast_to(b2, (16,))])

    BN = ACC_R // 16
    h1 = pl.pallas_call(
        _tc0_body,
        grid=(16,),
        in_specs=[
            pl.BlockSpec((BN, D), lambda i: (i, 0)),
            pl.BlockSpec((D, H), lambda i: (0, 0)),
        ],
        out_specs=pl.BlockSpec((BN, H), lambda i: (i, 0)),
        out_shape=jax.ShapeDtypeStruct((ACC_R, H), jnp.float32),
    )(x, W1)

    sc_all = _make_sc_all(NC, NS, CPW, ACC_R, H, N)
    out_flat, _, _, _, _ = sc_all(srcp, dstp, h1, consts, zeros1, zeros2)
    return out_flat[:N, None]
